# Initial kernel scaffold; baseline (speedup 1.0000x reference)
import jax
import jax.numpy as jnp
from jax.experimental import pallas as pl


def kernel(x, pe):
    raise NotImplementedError("write your pallas kernel here")


# speedup vs baseline: 3.2643x; 3.2643x over previous
import functools
import jax
import jax.numpy as jnp
from jax import lax
from jax.experimental import pallas as pl
from jax.experimental.pallas import tpu as pltpu, tpu_sc as plsc

D_MODEL = 64
NC, NS = 2, 16
NW = NC * NS
CHUNK = 128


def _gather_body(x_hbm, pe_hbm, out_hbm, idx_v, rows_v, gsem, osem,
                 *, b_per_w, nchunks):
    c = lax.axis_index("c")
    s = lax.axis_index("s")
    wid = s * NC + c
    base = wid * b_per_w

    def loop_body(j, carry):
        off = base + j * CHUNK
        pltpu.sync_copy(x_hbm.at[pl.ds(off, CHUNK)], idx_v)
        pltpu.async_copy(pe_hbm.at[idx_v], rows_v, gsem).wait()
        pltpu.async_copy(
            rows_v, out_hbm.at[pl.ds(off, CHUNK)], osem).wait()
        return carry

    lax.fori_loop(0, nchunks, loop_body, 0)


@jax.jit
def _pe_gather(y, pe):
    (b_total,) = y.shape
    b_per_w = b_total // NW
    nchunks = b_per_w // CHUNK
    mesh = plsc.VectorSubcoreMesh(
        core_axis_name="c", subcore_axis_name="s",
        num_cores=NC, num_subcores=NS)
    body = functools.partial(_gather_body, b_per_w=b_per_w, nchunks=nchunks)
    k = pl.kernel(
        body,
        out_type=jax.ShapeDtypeStruct((b_total, D_MODEL), jnp.float32),
        mesh=mesh,
        scratch_types=[
            pltpu.VMEM((CHUNK,), jnp.int32),
            pltpu.VMEM((CHUNK, D_MODEL), jnp.float32),
            pltpu.SemaphoreType.DMA,
            pltpu.SemaphoreType.DMA,
        ],
        compiler_params=pltpu.CompilerParams(use_tc_tiling_on_sc=False),
    )
    return k(y, pe)


def kernel(x, pe):
    bsize, sqlen = x.shape
    y = x.reshape(bsize * sqlen)
    out = _pe_gather(y, pe)
    return out.reshape(bsize, sqlen, D_MODEL)


# speedup vs baseline: 4.0112x; 1.2288x over previous
import functools
import jax
import jax.numpy as jnp
from jax import lax
from jax.experimental import pallas as pl
from jax.experimental.pallas import tpu as pltpu, tpu_sc as plsc

D_MODEL = 64
NC, NS = 2, 16
NW = NC * NS
CHUNK = 256
NBUF = 4


def _gather_body(x_hbm, pe_hbm, out_hbm, idx_v, rows_v, gsem, osem,
                 *, b_per_w, nchunks):
    c = lax.axis_index("c")
    s = lax.axis_index("s")
    wid = s * NC + c
    base = wid * b_per_w

    pltpu.sync_copy(x_hbm.at[pl.ds(base, b_per_w)], idx_v)

    def gather(j, b):
        return pltpu.async_copy(
            pe_hbm.at[idx_v.at[pl.ds(j * CHUNK, CHUNK)]],
            rows_v.at[b], gsem.at[b])

    def wait_gather(j, b):
        pltpu.make_async_copy(
            pe_hbm.at[idx_v.at[pl.ds(j * CHUNK, CHUNK)]],
            rows_v.at[b], gsem.at[b]).wait()

    def put(j, b):
        return pltpu.async_copy(
            rows_v.at[b], out_hbm.at[pl.ds(base + j * CHUNK, CHUNK)],
            osem.at[b])

    def wait_put(j, b):
        pltpu.make_async_copy(
            rows_v.at[b], out_hbm.at[pl.ds(base + j * CHUNK, CHUNK)],
            osem.at[b]).wait()

    ngroups = nchunks // NBUF

    for b in range(NBUF):
        gather(b, b)

    @pl.loop(0, ngroups)
    def _(g):
        j0 = g * NBUF
        for b in range(NBUF):
            j = j0 + b
            wait_gather(j, b)
            put(j, b)
            nj = j + NBUF

            @pl.when(nj < nchunks)
            def _():
                wait_put(j, b)
                gather(nj, b)

    for b in range(NBUF):
        wait_put(nchunks - NBUF + b, b)


@jax.jit
def _pe_gather(y, pe):
    (b_total,) = y.shape
    b_per_w = b_total // NW
    nchunks = b_per_w // CHUNK
    mesh = plsc.VectorSubcoreMesh(
        core_axis_name="c", subcore_axis_name="s",
        num_cores=NC, num_subcores=NS)
    body = functools.partial(_gather_body, b_per_w=b_per_w, nchunks=nchunks)
    k = pl.kernel(
        body,
        out_type=jax.ShapeDtypeStruct((b_total, D_MODEL), jnp.float32),
        mesh=mesh,
        scratch_types=[
            pltpu.VMEM((b_per_w,), jnp.int32),
            pltpu.VMEM((NBUF, CHUNK, D_MODEL), jnp.float32),
            pltpu.SemaphoreType.DMA((NBUF,)),
            pltpu.SemaphoreType.DMA((NBUF,)),
        ],
        compiler_params=pltpu.CompilerParams(use_tc_tiling_on_sc=False),
    )
    return k(y, pe)


def kernel(x, pe):
    bsize, sqlen = x.shape
    y = x.reshape(bsize * sqlen)
    out = _pe_gather(y, pe)
    return out.reshape(bsize, sqlen, D_MODEL)


# speedup vs baseline: 4.0270x; 1.0039x over previous
import functools
import jax
import jax.numpy as jnp
from jax import lax
from jax.experimental import pallas as pl
from jax.experimental.pallas import tpu as pltpu, tpu_sc as plsc

D_MODEL = 64
NC, NS = 2, 16
NW = NC * NS
CHUNK = 256
NBUF = 4


def _gather_body(x_hbm, pe_hbm, out_hbm, idx_v, rows_v, gsem, osem,
                 *, b_per_w, nchunks):
    c = lax.axis_index("c")
    s = lax.axis_index("s")
    wid = s * NC + c
    base = wid * b_per_w

    pltpu.sync_copy(x_hbm.at[pl.ds(base, b_per_w)], idx_v)

    def gather(j, b):
        return pltpu.async_copy(
            pe_hbm.at[idx_v.at[pl.ds(j * CHUNK, CHUNK)]],
            rows_v.at[b], gsem.at[b])

    def wait_gather(j, b):
        pltpu.make_async_copy(
            pe_hbm.at[idx_v.at[pl.ds(j * CHUNK, CHUNK)]],
            rows_v.at[b], gsem.at[b]).wait()

    def put(j, b):
        return pltpu.async_copy(
            rows_v.at[b], out_hbm.at[pl.ds(base + j * CHUNK, CHUNK)],
            osem.at[b])

    def wait_put(j, b):
        pltpu.make_async_copy(
            rows_v.at[b], out_hbm.at[pl.ds(base + j * CHUNK, CHUNK)],
            osem.at[b]).wait()

    L = 2
    ngroups = nchunks // NBUF

    for j in range(L):
        gather(j, j % NBUF)

    @pl.loop(0, ngroups)
    def _(g):
        j0 = g * NBUF
        for b in range(NBUF):
            j = j0 + b
            wait_gather(j, b)
            put(j, b)
            nj = j + L
            pj = j + L - NBUF

            @pl.when(nj < nchunks)
            def _():
                @pl.when(pj >= 0)
                def _():
                    wait_put(pj, nj % NBUF)
                gather(nj, nj % NBUF)

    for b in range(NBUF - L, NBUF):
        wait_put(nchunks - NBUF + b, b)
    for b in range(0, NBUF - L):
        wait_put(nchunks - NBUF + b, b)


@jax.jit
def _pe_gather(y, pe):
    (b_total,) = y.shape
    b_per_w = b_total // NW
    nchunks = b_per_w // CHUNK
    mesh = plsc.VectorSubcoreMesh(
        core_axis_name="c", subcore_axis_name="s",
        num_cores=NC, num_subcores=NS)
    body = functools.partial(_gather_body, b_per_w=b_per_w, nchunks=nchunks)
    k = pl.kernel(
        body,
        out_type=jax.ShapeDtypeStruct((b_total, D_MODEL), jnp.float32),
        mesh=mesh,
        scratch_types=[
            pltpu.VMEM((b_per_w,), jnp.int32),
            pltpu.VMEM((NBUF, CHUNK, D_MODEL), jnp.float32),
            pltpu.SemaphoreType.DMA((NBUF,)),
            pltpu.SemaphoreType.DMA((NBUF,)),
        ],
        compiler_params=pltpu.CompilerParams(use_tc_tiling_on_sc=False),
    )
    return k(y, pe)


def kernel(x, pe):
    bsize, sqlen = x.shape
    y = x.reshape(bsize * sqlen)
    out = _pe_gather(y, pe)
    return out.reshape(bsize, sqlen, D_MODEL)


# speedup vs baseline: 5.1110x; 1.2692x over previous
import functools
import jax
import jax.numpy as jnp
from jax import lax
from jax.experimental import pallas as pl
from jax.experimental.pallas import tpu as pltpu, tpu_sc as plsc

D_MODEL = 64
NC, NS = 2, 16
NW = NC * NS
CG = 8
SG = NW // CG
LANES = 16


def _gather_body(xT_hbm, peT_hbm, outT_hbm, table_v, idx_v, out_v, osem,
                 *, seqlen, batch, n_c):
    c = lax.axis_index("c")
    s = lax.axis_index("s")
    wid = s * NC + c
    c0 = (wid % CG) * n_c
    s_per = seqlen // SG
    s0 = (wid // CG) * s_per
    ngroups = batch // LANES

    pltpu.sync_copy(peT_hbm.at[pl.ds(c0, n_c)], table_v)

    def put(si, b):
        return pltpu.async_copy(
            out_v.at[b], outT_hbm.at[si, pl.ds(c0, n_c)], osem.at[b])

    def wait_put(si, b):
        pltpu.make_async_copy(
            out_v.at[b], outT_hbm.at[si, pl.ds(c0, n_c)], osem.at[b]).wait()

    @pl.loop(0, s_per, step=2)
    def _(i):
        for b in range(2):
            si = s0 + i + b

            @pl.when(i + b >= 2)
            def _():
                wait_put(si - 2, b)

            pltpu.sync_copy(xT_hbm.at[si], idx_v)

            @pl.loop(0, ngroups)
            def _(g):
                idx16 = idx_v[pl.ds(g * LANES, LANES)]
                for cl in range(n_c):
                    val = plsc.load_gather(
                        table_v,
                        [jnp.full((LANES,), cl, jnp.int32), idx16])
                    out_v[b, cl, pl.ds(g * LANES, LANES)] = val

            put(si, b)

    for b in range(2):
        wait_put(s0 + s_per - 2 + b, b)


@jax.jit
def _pe_gather(xT, peT):
    seqlen, batch = xT.shape
    n_c = peT.shape[0] // CG
    table_len = peT.shape[1]
    mesh = plsc.VectorSubcoreMesh(
        core_axis_name="c", subcore_axis_name="s",
        num_cores=NC, num_subcores=NS)
    body = functools.partial(_gather_body, seqlen=seqlen, batch=batch, n_c=n_c)
    k = pl.kernel(
        body,
        out_type=jax.ShapeDtypeStruct((seqlen, peT.shape[0], batch),
                                      jnp.float32),
        mesh=mesh,
        scratch_types=[
            pltpu.VMEM((n_c, table_len), jnp.float32),
            pltpu.VMEM((batch,), jnp.int32),
            pltpu.VMEM((2, n_c, batch), jnp.float32),
            pltpu.SemaphoreType.DMA((2,)),
        ],
        compiler_params=pltpu.CompilerParams(needs_layout_passes=False),
    )
    return k(xT, peT)


def kernel(x, pe):
    xT = jnp.swapaxes(x, 0, 1)
    peT = jnp.swapaxes(pe, 0, 1)
    outT = _pe_gather(xT, peT)
    return outT.transpose(2, 0, 1)


# speedup vs baseline: 5.3161x; 1.0401x over previous
import functools
import jax
import jax.numpy as jnp
from jax import lax
from jax.experimental import pallas as pl
from jax.experimental.pallas import tpu as pltpu, tpu_sc as plsc

D_MODEL = 64
NC, NS = 2, 16
NW = NC * NS
CG = 8
SG = NW // CG
LANES = 16


def _gather_body(xT_hbm, peT_hbm, outT_hbm, table_v, idx_v, out_v, osem,
                 *, seqlen, batch, n_c, table_len):
    c = lax.axis_index("c")
    s = lax.axis_index("s")
    wid = s * NC + c
    c0 = (wid % CG) * n_c
    s_per = seqlen // SG
    s0 = (wid // CG) * s_per
    ngroups = batch // LANES
    table_f = table_v

    for cl in range(n_c):
        pltpu.sync_copy(peT_hbm.at[c0 + cl],
                        table_v.at[pl.ds(cl * table_len, table_len)])

    def put(si, b):
        return pltpu.async_copy(
            out_v.at[b], outT_hbm.at[si, pl.ds(c0, n_c)], osem.at[b])

    def wait_put(si, b):
        pltpu.make_async_copy(
            out_v.at[b], outT_hbm.at[si, pl.ds(c0, n_c)], osem.at[b]).wait()

    @pl.loop(0, s_per, step=2)
    def _(i):
        for b in range(2):
            si = s0 + i + b

            @pl.when(i + b >= 2)
            def _():
                wait_put(si - 2, b)

            pltpu.sync_copy(xT_hbm.at[si], idx_v)

            @pl.loop(0, ngroups, unroll=4)
            def _(g):
                o = g * LANES
                idx16 = idx_v[pl.ds(o, LANES)]
                for cl in range(n_c):
                    val = plsc.load_gather(
                        table_f, [idx16 + jnp.int32(cl * table_len)])
                    out_v[b, cl, pl.ds(o, LANES)] = val

            put(si, b)

    for b in range(2):
        wait_put(s0 + s_per - 2 + b, b)


@jax.jit
def _pe_gather(xT, peT):
    seqlen, batch = xT.shape
    n_c = peT.shape[0] // CG
    table_len = peT.shape[1]
    mesh = plsc.VectorSubcoreMesh(
        core_axis_name="c", subcore_axis_name="s",
        num_cores=NC, num_subcores=NS)
    body = functools.partial(_gather_body, seqlen=seqlen, batch=batch,
                             n_c=n_c, table_len=table_len)
    k = pl.kernel(
        body,
        out_type=jax.ShapeDtypeStruct((seqlen, peT.shape[0], batch),
                                      jnp.float32),
        mesh=mesh,
        scratch_types=[
            pltpu.VMEM((n_c * table_len,), jnp.float32),
            pltpu.VMEM((batch,), jnp.int32),
            pltpu.VMEM((2, n_c, batch), jnp.float32),
            pltpu.SemaphoreType.DMA((2,)),
        ],
        compiler_params=pltpu.CompilerParams(needs_layout_passes=False),
    )
    return k(xT, peT)


def kernel(x, pe):
    xT = jnp.swapaxes(x, 0, 1)
    peT = jnp.swapaxes(pe, 0, 1)
    outT = _pe_gather(xT, peT)
    return outT.transpose(2, 0, 1)


# speedup vs baseline: 17.2432x; 3.2436x over previous
import functools
import jax
import jax.numpy as jnp
from jax import lax
from jax.experimental import pallas as pl
from jax.experimental.pallas import tpu as pltpu, tpu_sc as plsc

D_MODEL = 64
NC, NS = 2, 16
NW = NC * NS
CG = 8
SG = NW // CG
LANES = 16


def _gather_body(xT_hbm, peT_hbm, outT_hbm, table_v, idx_v, out_v, osem,
                 *, seqlen, batch, n_c, table_len):
    c = lax.axis_index("c")
    s = lax.axis_index("s")
    wid = s * NC + c
    c0 = (wid % CG) * n_c
    s_per = seqlen // SG
    s0 = (wid // CG) * s_per
    ngroups = batch // LANES
    table_f = table_v

    for cl in range(n_c):
        pltpu.sync_copy(peT_hbm.at[c0 + cl],
                        table_v.at[pl.ds(cl * table_len, table_len)])

    def put(si, b):
        return pltpu.async_copy(
            out_v.at[b], outT_hbm.at[si, pl.ds(c0, n_c)], osem.at[b])

    def wait_put(si, b):
        pltpu.make_async_copy(
            out_v.at[b], outT_hbm.at[si, pl.ds(c0, n_c)], osem.at[b]).wait()

    @pl.loop(0, s_per, step=2)
    def _(i):
        for b in range(2):
            si = s0 + i + b

            @pl.when(i + b >= 2)
            def _():
                wait_put(si - 2, b)

            pltpu.sync_copy(xT_hbm.at[si], idx_v)

            @plsc.parallel_loop(0, batch, step=LANES, unroll=4)
            def _(o):
                idx16 = idx_v[pl.ds(o, LANES)]
                for cl in range(n_c):
                    val = plsc.load_gather(
                        table_f, [idx16 + jnp.int32(cl * table_len)])
                    out_v[b, cl, pl.ds(o, LANES)] = val

            put(si, b)

    for b in range(2):
        wait_put(s0 + s_per - 2 + b, b)


@jax.jit
def _pe_gather(xT, peT):
    seqlen, batch = xT.shape
    n_c = peT.shape[0] // CG
    table_len = peT.shape[1]
    mesh = plsc.VectorSubcoreMesh(
        core_axis_name="c", subcore_axis_name="s",
        num_cores=NC, num_subcores=NS)
    body = functools.partial(_gather_body, seqlen=seqlen, batch=batch,
                             n_c=n_c, table_len=table_len)
    k = pl.kernel(
        body,
        out_type=jax.ShapeDtypeStruct((seqlen, peT.shape[0], batch),
                                      jnp.float32),
        mesh=mesh,
        scratch_types=[
            pltpu.VMEM((n_c * table_len,), jnp.float32),
            pltpu.VMEM((batch,), jnp.int32),
            pltpu.VMEM((2, n_c, batch), jnp.float32),
            pltpu.SemaphoreType.DMA((2,)),
        ],
        compiler_params=pltpu.CompilerParams(needs_layout_passes=False),
    )
    return k(xT, peT)


def kernel(x, pe):
    xT = jnp.swapaxes(x, 0, 1)
    peT = jnp.swapaxes(pe, 0, 1)
    outT = _pe_gather(xT, peT)
    return outT.transpose(2, 0, 1)


# speedup vs baseline: 17.3077x; 1.0037x over previous
import functools
import jax
import jax.numpy as jnp
from jax import lax
from jax.experimental import pallas as pl
from jax.experimental.pallas import tpu as pltpu, tpu_sc as plsc

D_MODEL = 64
NC, NS = 2, 16
NW = NC * NS
CG = 8
SG = NW // CG
LANES = 16


def _gather_body(xT_hbm, peT_hbm, outT_hbm, table_v, idx_v, out_v, osem,
                 *, seqlen, batch, n_c, table_len):
    c = lax.axis_index("c")
    s = lax.axis_index("s")
    wid = s * NC + c
    c0 = (wid % CG) * n_c
    s_per = seqlen // SG
    s0 = (wid // CG) * s_per
    ngroups = batch // LANES
    table_f = table_v

    for cl in range(n_c):
        pltpu.sync_copy(peT_hbm.at[c0 + cl],
                        table_v.at[pl.ds(cl * table_len, table_len)])

    def put(si, b):
        return pltpu.async_copy(
            out_v.at[b], outT_hbm.at[si, pl.ds(c0, n_c)], osem.at[b])

    def wait_put(si, b):
        pltpu.make_async_copy(
            out_v.at[b], outT_hbm.at[si, pl.ds(c0, n_c)], osem.at[b]).wait()

    @pl.loop(0, s_per, step=2)
    def _(i):
        for b in range(2):
            si = s0 + i + b

            @pl.when(i + b >= 2)
            def _():
                wait_put(si - 2, b)

            pltpu.sync_copy(xT_hbm.at[si], idx_v)

            @plsc.parallel_loop(0, batch, step=LANES, unroll=8)
            def _(o):
                idx16 = idx_v[pl.ds(o, LANES)]
                for cl in range(n_c):
                    val = plsc.load_gather(
                        table_f, [idx16 + jnp.int32(cl * table_len)])
                    out_v[b, cl, pl.ds(o, LANES)] = val

            put(si, b)

    for b in range(2):
        wait_put(s0 + s_per - 2 + b, b)


@jax.jit
def _pe_gather(xT, peT):
    seqlen, batch = xT.shape
    n_c = peT.shape[0] // CG
    table_len = peT.shape[1]
    mesh = plsc.VectorSubcoreMesh(
        core_axis_name="c", subcore_axis_name="s",
        num_cores=NC, num_subcores=NS)
    body = functools.partial(_gather_body, seqlen=seqlen, batch=batch,
                             n_c=n_c, table_len=table_len)
    k = pl.kernel(
        body,
        out_type=jax.ShapeDtypeStruct((seqlen, peT.shape[0], batch),
                                      jnp.float32),
        mesh=mesh,
        scratch_types=[
            pltpu.VMEM((n_c * table_len,), jnp.float32),
            pltpu.VMEM((batch,), jnp.int32),
            pltpu.VMEM((2, n_c, batch), jnp.float32),
            pltpu.SemaphoreType.DMA((2,)),
        ],
        compiler_params=pltpu.CompilerParams(needs_layout_passes=False),
    )
    return k(xT, peT)


def kernel(x, pe):
    xT = jnp.swapaxes(x, 0, 1)
    peT = jnp.swapaxes(pe, 0, 1)
    outT = _pe_gather(xT, peT)
    return outT.transpose(2, 0, 1)


# speedup vs baseline: 22.9978x; 1.3288x over previous
import functools
import jax
import jax.numpy as jnp
from jax import lax
from jax.experimental import pallas as pl
from jax.experimental.pallas import tpu as pltpu, tpu_sc as plsc

D_MODEL = 64
NC, NS = 2, 16
NW = NC * NS
CG = 8
SG = NW // CG
LANES = 16


def _gather_body(xT_hbm, peT_hbm, outT_hbm, table_v, idx_v, out_v, osem, isem,
                 *, seqlen, batch, n_c, table_len):
    c = lax.axis_index("c")
    s = lax.axis_index("s")
    wid = s * NC + c
    c0 = (wid % CG) * n_c
    s_per = seqlen // SG
    s0 = (wid // CG) * s_per
    ngroups = batch // LANES
    table_f = table_v

    for cl in range(n_c):
        pltpu.sync_copy(peT_hbm.at[c0 + cl],
                        table_v.at[pl.ds(cl * table_len, table_len)])

    def put(si, b):
        return pltpu.async_copy(
            out_v.at[b], outT_hbm.at[si, pl.ds(c0, n_c)], osem.at[b])

    def wait_put(si, b):
        pltpu.make_async_copy(
            out_v.at[b], outT_hbm.at[si, pl.ds(c0, n_c)], osem.at[b]).wait()

    def get_idx(si, ib):
        return pltpu.async_copy(xT_hbm.at[si], idx_v.at[ib], isem.at[ib])

    def wait_idx(si, ib):
        pltpu.make_async_copy(xT_hbm.at[si], idx_v.at[ib], isem.at[ib]).wait()

    get_idx(s0, 0)

    @pl.loop(0, s_per, step=2)
    def _(i):
        for b in range(2):
            si = s0 + i + b
            wait_idx(si, b)

            @pl.when(i + b + 1 < s_per)
            def _():
                get_idx(si + 1, 1 - b)

            @pl.when(i + b >= 2)
            def _():
                wait_put(si - 2, b)

            @plsc.parallel_loop(0, batch, step=LANES, unroll=8)
            def _(o):
                idx16 = idx_v[b, pl.ds(o, LANES)]
                for cl in range(n_c):
                    val = plsc.load_gather(
                        table_f, [idx16 + jnp.int32(cl * table_len)])
                    out_v[b, cl, pl.ds(o, LANES)] = val

            put(si, b)

    for b in range(2):
        wait_put(s0 + s_per - 2 + b, b)


@jax.jit
def _pe_gather(xT, peT):
    seqlen, batch = xT.shape
    n_c = peT.shape[0] // CG
    table_len = peT.shape[1]
    mesh = plsc.VectorSubcoreMesh(
        core_axis_name="c", subcore_axis_name="s",
        num_cores=NC, num_subcores=NS)
    body = functools.partial(_gather_body, seqlen=seqlen, batch=batch,
                             n_c=n_c, table_len=table_len)
    k = pl.kernel(
        body,
        out_type=jax.ShapeDtypeStruct((seqlen, peT.shape[0], batch),
                                      jnp.float32),
        mesh=mesh,
        scratch_types=[
            pltpu.VMEM((n_c * table_len,), jnp.float32),
            pltpu.VMEM((2, batch), jnp.int32),
            pltpu.VMEM((2, n_c, batch), jnp.float32),
            pltpu.SemaphoreType.DMA((2,)),
            pltpu.SemaphoreType.DMA((2,)),
        ],
        compiler_params=pltpu.CompilerParams(needs_layout_passes=False),
    )
    return k(xT, peT)


def kernel(x, pe):
    xT = jnp.swapaxes(x, 0, 1)
    peT = jnp.swapaxes(pe, 0, 1)
    outT = _pe_gather(xT, peT)
    return outT.transpose(2, 0, 1)
